# baseline (speedup 1.0000x reference)
import functools

import jax
import jax.numpy as jnp
from jax import lax
from jax.experimental import pallas as pl
from jax.experimental.pallas import tpu as pltpu
from jax.experimental.pallas import tpu_sc as plsc

NUM_CODES_K = 1024
DIM_K = 64
ROWS = 9216
ROW_BLOCK = 1152
NBLK = ROWS // ROW_BLOCK

NW = 32
B_PER_W = ROWS // NW
CHUNKS = 3
CHUNK = B_PER_W // CHUNKS


def _dist_argmin_body(xt_ref, cb_ref, idx_ref, acc_ref):
    xt = xt_ref[...]
    cb = cb_ref[...]
    mm = lax.dot_general(cb, xt, (((1,), (0,)), ((), ())),
                         preferred_element_type=jnp.float32)
    x2 = jnp.sum(xt * xt, axis=0, keepdims=True)
    c2 = jnp.sum(cb * cb, axis=1, keepdims=True)
    d = (x2 + c2) - 2.0 * mm
    m = jnp.min(d, axis=0, keepdims=True)
    iota = lax.broadcasted_iota(jnp.int32, d.shape, 0).astype(jnp.float32)
    idx = jnp.min(jnp.where(d == m, iota, float(NUM_CODES_K)), axis=0)
    idx_ref[0, 0, :] = idx.astype(jnp.int32)

    @pl.when(pl.program_id(0) == 0)
    def _():
        acc_ref[0, 0] = 0.0

    acc_ref[0, 0] += jnp.sum(m)


def _dist_argmin(xt, cb, start, nblk):
    return pl.pallas_call(
        _dist_argmin_body,
        grid=(nblk,),
        in_specs=[
            pl.BlockSpec((DIM_K, ROW_BLOCK), lambda i: (0, i + start)),
            pl.BlockSpec((NUM_CODES_K, DIM_K), lambda i: (0, 0)),
        ],
        out_specs=[
            pl.BlockSpec((1, 1, ROW_BLOCK), lambda i: (i, 0, 0)),
            pl.BlockSpec(memory_space=pltpu.SMEM),
        ],
        out_shape=[
            jax.ShapeDtypeStruct((nblk, 1, ROW_BLOCK), jnp.int32),
            jax.ShapeDtypeStruct((1, 1), jnp.float32),
        ],
    )(xt, cb)


@functools.cache
def _make_sc_gather(b_per_w, chunks, chunk):
    @functools.partial(
        pl.kernel,
        mesh=plsc.VectorSubcoreMesh(core_axis_name="c", subcore_axis_name="s"),
        out_type=jax.ShapeDtypeStruct((NW, b_per_w, DIM_K), jnp.float32),
        scratch_types=[
            pltpu.VMEM((chunks, chunk), jnp.int32),
            pltpu.VMEM((b_per_w, DIM_K), jnp.float32),
            pltpu.SemaphoreType.DMA,
        ],
        compiler_params=pltpu.CompilerParams(use_tc_tiling_on_sc=False),
    )
    def _sc_gather(cb_hbm, idx_hbm, out_hbm, idx_v, rows_v, sem):
        wid = lax.axis_index("s") * 2 + lax.axis_index("c")
        pltpu.sync_copy(idx_hbm.at[wid], idx_v)
        copies = [
            pltpu.async_copy(cb_hbm.at[idx_v.at[k]],
                             rows_v.at[pl.ds(k * chunk, chunk)], sem)
            for k in range(chunks)
        ]
        for cp in copies:
            cp.wait()
        pltpu.sync_copy(rows_v, out_hbm.at[wid])

    return _sc_gather


HALF_BLK = NBLK // 2
HALF_ROWS = ROWS // 2
HB_PER_W = HALF_ROWS // NW
HCHUNKS = 2
HCHUNK = HB_PER_W // HCHUNKS


def kernel(inputs, codebook):
    xt = inputs.reshape(-1, DIM_K).T
    gather = _make_sc_gather(HB_PER_W, HCHUNKS, HCHUNK)
    idx_a, acc_a = _dist_argmin(xt, codebook, 0, HALF_BLK)
    q_a = gather(codebook, idx_a.reshape(NW, HCHUNKS, HCHUNK))
    idx_b, acc_b = _dist_argmin(xt, codebook, HALF_BLK, HALF_BLK)
    q_b = gather(codebook, idx_b.reshape(NW, HCHUNKS, HCHUNK))
    quantized = jnp.concatenate(
        [q_a.reshape(HALF_ROWS, DIM_K), q_b.reshape(HALF_ROWS, DIM_K)]
    ).reshape(inputs.shape)
    loss = (acc_a[0, 0] + acc_b[0, 0]) / float(ROWS * DIM_K)
    enc = jnp.concatenate(
        [idx_a.reshape(HALF_ROWS), idx_b.reshape(HALF_ROWS)]
    ).reshape(inputs.shape[:-1])
    return (quantized, loss, loss, quantized, enc)
